# Initial kernel scaffold; baseline (speedup 1.0000x reference)
import jax
import jax.numpy as jnp
from jax.experimental import pallas as pl


def kernel(green_pred, xtrans, chroma_pred):
    raise NotImplementedError("write your pallas kernel here")


# speedup vs baseline: 294.5363x; 294.5363x over previous
import numpy as np
import jax
import jax.numpy as jnp
from jax.experimental import pallas as pl

_FACTOR = 6
_R_POS = [(0, 4), (1, 0), (1, 2), (2, 4), (3, 1), (4, 3), (4, 5), (5, 1)]
_B_POS = [(0, 1), (1, 3), (1, 5), (2, 1), (3, 4), (4, 0), (4, 2), (5, 4)]


def _mask6(pos_list):
    m = np.zeros((_FACTOR, _FACTOR), dtype=bool)
    for r, c in pos_list:
        m[r, c] = True
    return m


def _select_kernel(g_ref, x0_ref, x2_ref, c_ref, m0_ref, m2_ref, o_ref):
    o_ref[0, 0, :, :] = jnp.where(m0_ref[...] != 0, x0_ref[0, 0], c_ref[0, 0])
    o_ref[0, 1, :, :] = g_ref[0, 0]
    o_ref[0, 2, :, :] = jnp.where(m2_ref[...] != 0, x2_ref[0, 0], c_ref[0, 1])


def kernel(green_pred, xtrans, chroma_pred):
    B, _, H, W = green_pred.shape
    HB = 192
    assert H % HB == 0 and W % _FACTOR == 0

    reps = (HB // _FACTOR, W // _FACTOR)
    keep0 = jnp.asarray(np.tile(_mask6(_R_POS), reps).astype(np.int32))
    keep2 = jnp.asarray(np.tile(_mask6(_B_POS), reps).astype(np.int32))

    grid = (B, H // HB)
    img_spec = lambda c: pl.BlockSpec((1, 1, HB, W), lambda b, h, c=c: (b, c, h, 0))
    mask_spec = pl.BlockSpec((HB, W), lambda b, h: (0, 0))

    return pl.pallas_call(
        _select_kernel,
        grid=grid,
        in_specs=[
            img_spec(0),
            img_spec(0),
            img_spec(2),
            pl.BlockSpec((1, 2, HB, W), lambda b, h: (b, 0, h, 0)),
            mask_spec,
            mask_spec,
        ],
        out_specs=pl.BlockSpec((1, 3, HB, W), lambda b, h: (b, 0, h, 0)),
        out_shape=jax.ShapeDtypeStruct((B, 3, H, W), green_pred.dtype),
    )(green_pred, xtrans, xtrans, chroma_pred, keep0, keep2)


# speedup vs baseline: 295.0686x; 1.0018x over previous
import numpy as np
import jax
import jax.numpy as jnp
from jax.experimental import pallas as pl

_FACTOR = 6
_R_POS = [(0, 4), (1, 0), (1, 2), (2, 4), (3, 1), (4, 3), (4, 5), (5, 1)]
_B_POS = [(0, 1), (1, 3), (1, 5), (2, 1), (3, 4), (4, 0), (4, 2), (5, 4)]


def _mask6(pos_list):
    m = np.zeros((_FACTOR, _FACTOR), dtype=bool)
    for r, c in pos_list:
        m[r, c] = True
    return m


def _select_kernel(g_ref, x0_ref, x2_ref, c_ref, m0_ref, m2_ref, o_ref):
    o_ref[0, 0, :, :] = jnp.where(m0_ref[...] != 0, x0_ref[0, 0], c_ref[0, 0])
    o_ref[0, 1, :, :] = g_ref[0, 0]
    o_ref[0, 2, :, :] = jnp.where(m2_ref[...] != 0, x2_ref[0, 0], c_ref[0, 1])


def kernel(green_pred, xtrans, chroma_pred):
    B, _, H, W = green_pred.shape
    HB = 384
    assert H % HB == 0 and W % _FACTOR == 0

    reps = (HB // _FACTOR, W // _FACTOR)
    keep0 = jnp.asarray(np.tile(_mask6(_R_POS), reps).astype(np.int32))
    keep2 = jnp.asarray(np.tile(_mask6(_B_POS), reps).astype(np.int32))

    grid = (B, H // HB)
    img_spec = lambda c: pl.BlockSpec((1, 1, HB, W), lambda b, h, c=c: (b, c, h, 0))
    mask_spec = pl.BlockSpec((HB, W), lambda b, h: (0, 0))

    return pl.pallas_call(
        _select_kernel,
        grid=grid,
        in_specs=[
            img_spec(0),
            img_spec(0),
            img_spec(2),
            pl.BlockSpec((1, 2, HB, W), lambda b, h: (b, 0, h, 0)),
            mask_spec,
            mask_spec,
        ],
        out_specs=pl.BlockSpec((1, 3, HB, W), lambda b, h: (b, 0, h, 0)),
        out_shape=jax.ShapeDtypeStruct((B, 3, H, W), green_pred.dtype),
    )(green_pred, xtrans, xtrans, chroma_pred, keep0, keep2)
